# baseline (device time: 304511 ns/iter reference)
import jax
import jax.numpy as jnp
from jax import lax
from jax.experimental import pallas as pl
from jax.experimental.pallas import tpu as pltpu

N_DEV = 8
SQ = 512
D = 1024
HL = 8
DH = 128
SKV = 2048
SCALE = 0.08838834764831843
F32 = jnp.float32
BF16 = jnp.bfloat16


def kernel(x, Wq, Wo, K_ext, V_ext):
    x2 = x.reshape(SQ, D).astype(BF16)
    wq3 = (Wq * SCALE).reshape(D, HL, DH).transpose(1, 0, 2).astype(BF16)
    wo3 = Wo.reshape(HL, DH, D).astype(BF16)

    def body(x_ref, wq_ref, wo_ref, k_hbm, v_hbm, out_ref,
             k_vmem, v_vmem, k_f32, v_f32, xslot, rs_send_buf, rs_recv_buf,
             pacc, kv_sems, x_send_sems, x_recv_sems, rs_send_sems,
             rs_recv_sems, x_credit, rs_credit):
        my = lax.axis_index("i")
        right = (my + 1) % N_DEV
        left = (my + N_DEV - 1) % N_DEV
        h0 = my * HL

        kv_copies = []
        for h in range(HL):
            kcp = pltpu.make_async_copy(
                k_hbm.at[0, :, h0 + h, :], k_f32.at[h], kv_sems.at[h])
            vcp = pltpu.make_async_copy(
                v_hbm.at[0, :, h0 + h, :], v_f32.at[h], kv_sems.at[HL + h])
            kcp.start()
            vcp.start()
            kv_copies.append((kcp, vcp))

        barrier = pltpu.get_barrier_semaphore()
        for nbr in (left, right):
            pl.semaphore_signal(barrier, inc=1, device_id=(nbr,),
                                device_id_type=pl.DeviceIdType.MESH)
        pl.semaphore_wait(barrier, 2)

        for kcp, vcp in kv_copies:
            kcp.wait()
            vcp.wait()
        k_vmem[:, :, :] = k_f32[:, :, :].astype(BF16)
        v_vmem[:, :, :] = v_f32[:, :, :].astype(BF16)

        def compute_chunk(src_ref, src_off, dst_ref, dst_off):
            pacc[:, :] = jnp.zeros((SQ, D), F32)

            def head_step(h, carry):
                xc = src_ref[pl.ds(src_off, SQ), :]
                qh = jnp.dot(xc, wq_ref[h], preferred_element_type=F32)
                s = lax.dot_general(
                    qh.astype(BF16), k_vmem[h], (((1,), (1,)), ((), ())),
                    preferred_element_type=F32)
                p = jnp.exp(s.astype(BF16))
                l = jnp.sum(p, axis=1, keepdims=True,
                            dtype=F32)
                oh = jnp.dot(p, v_vmem[h],
                             preferred_element_type=F32) / l
                pacc[:, :] = pacc[:, :] + jnp.dot(
                    oh.astype(BF16), wo_ref[h], preferred_element_type=F32)
                return carry

            lax.fori_loop(0, HL, head_step, None)
            dst_ref[pl.ds(dst_off, SQ), :] = pacc[:, :].astype(dst_ref.dtype)

        def xslot_at(p):
            return xslot.at[pl.ds(p * SQ, SQ), :]

        def rs_send_at(p):
            return rs_send_buf.at[pl.ds(p * SQ, SQ), :]

        def rs_recv_at(p):
            return rs_recv_buf.at[pl.ds(p * SQ, SQ), :]

        xslot[pl.ds(0, SQ), :] = x_ref[:, :]
        compute_chunk(x_ref, 0, out_ref, 0)

        def step(t, _):
            sp = (t - 1) % 2
            dp = t % 2

            @pl.when(t >= 2)
            def _():
                pl.semaphore_wait(x_credit.at[dp], 1)

            xr = pltpu.make_async_remote_copy(
                src_ref=xslot_at(sp),
                dst_ref=xslot_at(dp),
                send_sem=x_send_sems.at[sp],
                recv_sem=x_recv_sems.at[dp],
                device_id=(right,),
                device_id_type=pl.DeviceIdType.MESH,
            )
            xr.start()

            @pl.when(t >= 2)
            def _():
                @pl.when(t >= 4)
                def _():
                    pl.semaphore_wait(rs_credit.at[dp], 1)

                @pl.when(t >= 3)
                def _():
                    rs_in = pltpu.make_async_remote_copy(
                        src_ref=rs_send_at(sp),
                        dst_ref=rs_recv_at(sp),
                        send_sem=rs_send_sems.at[sp],
                        recv_sem=rs_recv_sems.at[sp],
                        device_id=(left,),
                        device_id_type=pl.DeviceIdType.MESH,
                    )
                    rs_in.wait_recv()
                    rs_send_buf[pl.ds(sp * SQ, SQ), :] = (
                        rs_send_buf[pl.ds(sp * SQ, SQ), :]
                        + rs_recv_buf[pl.ds(sp * SQ, SQ), :])
                    pl.semaphore_signal(rs_credit.at[sp], inc=1,
                                        device_id=(left,),
                                        device_id_type=pl.DeviceIdType.MESH)

                rs_out = pltpu.make_async_remote_copy(
                    src_ref=rs_send_at(sp),
                    dst_ref=rs_recv_at(dp),
                    send_sem=rs_send_sems.at[sp],
                    recv_sem=rs_recv_sems.at[dp],
                    device_id=(right,),
                    device_id_type=pl.DeviceIdType.MESH,
                )
                rs_out.start()

            xr.wait_recv()
            compute_chunk(xslot, dp * SQ, rs_send_buf, dp * SQ)

            xr.wait_send()

            @pl.when(t <= 6)
            def _():
                pl.semaphore_signal(x_credit.at[sp], inc=1,
                                    device_id=(left,),
                                    device_id_type=pl.DeviceIdType.MESH)

            @pl.when(t >= 2)
            def _():
                rs_done = pltpu.make_async_remote_copy(
                    src_ref=rs_send_at(sp),
                    dst_ref=rs_recv_at(dp),
                    send_sem=rs_send_sems.at[sp],
                    recv_sem=rs_recv_sems.at[dp],
                    device_id=(right,),
                    device_id_type=pl.DeviceIdType.MESH,
                )
                rs_done.wait_send()

            return _

        lax.fori_loop(1, N_DEV, step, None)

        pl.semaphore_wait(rs_credit.at[0], 1)
        rs_in = pltpu.make_async_remote_copy(
            src_ref=rs_send_at(1), dst_ref=rs_recv_at(1),
            send_sem=rs_send_sems.at[1], recv_sem=rs_recv_sems.at[1],
            device_id=(left,), device_id_type=pl.DeviceIdType.MESH,
        )
        rs_in.wait_recv()
        rs_send_buf[pl.ds(SQ, SQ), :] = (
            rs_send_buf[pl.ds(SQ, SQ), :] + rs_recv_buf[pl.ds(SQ, SQ), :])
        rs_out = pltpu.make_async_remote_copy(
            src_ref=rs_send_at(1), dst_ref=rs_recv_at(0),
            send_sem=rs_send_sems.at[1], recv_sem=rs_recv_sems.at[0],
            device_id=(right,), device_id_type=pl.DeviceIdType.MESH,
        )
        rs_out.start()
        rs_out.wait_recv()
        out_ref[:, :] = out_ref[:, :] + rs_recv_buf[pl.ds(0, SQ), :]
        rs_out.wait_send()

    out = pl.pallas_call(
        body,
        out_shape=jax.ShapeDtypeStruct((SQ, D), F32),
        in_specs=[
            pl.BlockSpec(memory_space=pltpu.MemorySpace.VMEM),
            pl.BlockSpec(memory_space=pltpu.MemorySpace.VMEM),
            pl.BlockSpec(memory_space=pltpu.MemorySpace.VMEM),
            pl.BlockSpec(memory_space=pl.ANY),
            pl.BlockSpec(memory_space=pl.ANY),
        ],
        out_specs=pl.BlockSpec(memory_space=pltpu.MemorySpace.VMEM),
        scratch_shapes=[
            pltpu.VMEM((HL, SKV, DH), BF16),
            pltpu.VMEM((HL, SKV, DH), BF16),
            pltpu.VMEM((HL, SKV, DH), F32),
            pltpu.VMEM((HL, SKV, DH), F32),
            pltpu.VMEM((2 * SQ, D), BF16),
            pltpu.VMEM((2 * SQ, D), BF16),
            pltpu.VMEM((2 * SQ, D), BF16),
            pltpu.VMEM((SQ, D), F32),
            pltpu.SemaphoreType.DMA((2 * HL,)),
            pltpu.SemaphoreType.DMA((2,)),
            pltpu.SemaphoreType.DMA((2,)),
            pltpu.SemaphoreType.DMA((2,)),
            pltpu.SemaphoreType.DMA((2,)),
            pltpu.SemaphoreType.REGULAR((2,)),
            pltpu.SemaphoreType.REGULAR((2,)),
        ],
        compiler_params=pltpu.CompilerParams(
            collective_id=0,
            vmem_limit_bytes=60 * 1024 * 1024,
        ),
    )(x2, wq3, wo3, K_ext, V_ext)
    return out.reshape(1, SQ, D)


# device time: 283617 ns/iter; 1.0737x vs baseline; 1.0737x over previous
import jax
import jax.numpy as jnp
from jax import lax
from jax.experimental import pallas as pl
from jax.experimental.pallas import tpu as pltpu

N_DEV = 8
SQ = 512
D = 1024
HL = 8
DH = 128
SKV = 2048
SCALE = 0.08838834764831843
F32 = jnp.float32
BF16 = jnp.bfloat16


def kernel(x, Wq, Wo, K_ext, V_ext):
    x2 = x.reshape(SQ, D).astype(BF16)
    wq3 = (Wq * SCALE).reshape(D, HL, DH).transpose(1, 0, 2).astype(BF16)
    wo3 = Wo.reshape(HL, DH, D).astype(BF16)

    def body(x_ref, wq_ref, wo_ref, k_hbm, v_hbm, out_ref,
             k_vmem, v_vmem, k_f32, v_f32, xslot, rs_send_buf, rs_recv_buf,
             pacc, kv_sems, x_send_sems, x_recv_sems, rs_send_sems,
             rs_recv_sems, x_credit, rs_credit):
        my = lax.axis_index("i")
        right = (my + 1) % N_DEV
        left = (my + N_DEV - 1) % N_DEV
        h0 = my * HL

        kv_copies = []
        for h in range(HL):
            kcp = pltpu.make_async_copy(
                k_hbm.at[0, :, h0 + h, :], k_f32.at[h], kv_sems.at[h])
            vcp = pltpu.make_async_copy(
                v_hbm.at[0, :, h0 + h, :], v_f32.at[h], kv_sems.at[HL + h])
            kcp.start()
            vcp.start()
            kv_copies.append((kcp, vcp))

        barrier = pltpu.get_barrier_semaphore()
        for nbr in (left, right):
            pl.semaphore_signal(barrier, inc=1, device_id=(nbr,),
                                device_id_type=pl.DeviceIdType.MESH)
        pl.semaphore_wait(barrier, 2)

        for kcp, vcp in kv_copies:
            kcp.wait()
            vcp.wait()
        k_vmem[:, :, :] = k_f32[:, :, :].astype(BF16)
        v_vmem[:, :, :] = v_f32[:, :, :].astype(BF16)

        def compute_chunk(src_ref, src_off, dst_ref, dst_off):
            pacc[:, :] = jnp.zeros((SQ, D), F32)

            def head_step(h, carry):
                xc = src_ref[pl.ds(src_off, SQ), :]
                qh = jnp.dot(xc, wq_ref[h], preferred_element_type=F32)
                s = lax.dot_general(
                    qh.astype(BF16), k_vmem[h], (((1,), (1,)), ((), ())),
                    preferred_element_type=F32)
                p = jnp.exp(s)
                l = jnp.sum(p, axis=1, keepdims=True)
                oh = jnp.dot(p.astype(BF16), v_vmem[h],
                             preferred_element_type=F32) / l
                pacc[:, :] = pacc[:, :] + jnp.dot(
                    oh.astype(BF16), wo_ref[h], preferred_element_type=F32)
                return carry

            lax.fori_loop(0, HL, head_step, None)
            dst_ref[pl.ds(dst_off, SQ), :] = pacc[:, :].astype(dst_ref.dtype)

        def xslot_at(p):
            return xslot.at[pl.ds(p * SQ, SQ), :]

        def rs_send_at(p):
            return rs_send_buf.at[pl.ds(p * SQ, SQ), :]

        def rs_recv_at(p):
            return rs_recv_buf.at[pl.ds(p * SQ, SQ), :]

        xslot[pl.ds(0, SQ), :] = x_ref[:, :]
        xr1 = pltpu.make_async_remote_copy(
            src_ref=xslot_at(0), dst_ref=xslot_at(1),
            send_sem=x_send_sems.at[0], recv_sem=x_recv_sems.at[1],
            device_id=(right,), device_id_type=pl.DeviceIdType.MESH,
        )
        xr1.start()
        compute_chunk(x_ref, 0, out_ref, 0)

        def step(t, _):
            sp = (t - 1) % 2
            dp = t % 2

            @pl.when(t >= 2)
            def _():
                pl.semaphore_wait(x_credit.at[dp], 1)

            xr = pltpu.make_async_remote_copy(
                src_ref=xslot_at(sp),
                dst_ref=xslot_at(dp),
                send_sem=x_send_sems.at[sp],
                recv_sem=x_recv_sems.at[dp],
                device_id=(right,),
                device_id_type=pl.DeviceIdType.MESH,
            )

            @pl.when(t >= 2)
            def _():
                xr.start()

            @pl.when(t >= 2)
            def _():
                @pl.when(t >= 4)
                def _():
                    pl.semaphore_wait(rs_credit.at[dp], 1)

                @pl.when(t >= 3)
                def _():
                    rs_in = pltpu.make_async_remote_copy(
                        src_ref=rs_send_at(sp),
                        dst_ref=rs_recv_at(sp),
                        send_sem=rs_send_sems.at[sp],
                        recv_sem=rs_recv_sems.at[sp],
                        device_id=(left,),
                        device_id_type=pl.DeviceIdType.MESH,
                    )
                    rs_in.wait_recv()
                    rs_send_buf[pl.ds(sp * SQ, SQ), :] = (
                        rs_send_buf[pl.ds(sp * SQ, SQ), :]
                        + rs_recv_buf[pl.ds(sp * SQ, SQ), :])
                    pl.semaphore_signal(rs_credit.at[sp], inc=1,
                                        device_id=(left,),
                                        device_id_type=pl.DeviceIdType.MESH)

                rs_out = pltpu.make_async_remote_copy(
                    src_ref=rs_send_at(sp),
                    dst_ref=rs_recv_at(dp),
                    send_sem=rs_send_sems.at[sp],
                    recv_sem=rs_recv_sems.at[dp],
                    device_id=(right,),
                    device_id_type=pl.DeviceIdType.MESH,
                )
                rs_out.start()

            xr.wait_recv()
            compute_chunk(xslot, dp * SQ, rs_send_buf, dp * SQ)

            xr.wait_send()

            @pl.when(t <= 6)
            def _():
                pl.semaphore_signal(x_credit.at[sp], inc=1,
                                    device_id=(left,),
                                    device_id_type=pl.DeviceIdType.MESH)

            @pl.when(t >= 2)
            def _():
                rs_done = pltpu.make_async_remote_copy(
                    src_ref=rs_send_at(sp),
                    dst_ref=rs_recv_at(dp),
                    send_sem=rs_send_sems.at[sp],
                    recv_sem=rs_recv_sems.at[dp],
                    device_id=(right,),
                    device_id_type=pl.DeviceIdType.MESH,
                )
                rs_done.wait_send()

            return _

        lax.fori_loop(1, N_DEV, step, None)

        pl.semaphore_wait(rs_credit.at[0], 1)
        rs_in = pltpu.make_async_remote_copy(
            src_ref=rs_send_at(1), dst_ref=rs_recv_at(1),
            send_sem=rs_send_sems.at[1], recv_sem=rs_recv_sems.at[1],
            device_id=(left,), device_id_type=pl.DeviceIdType.MESH,
        )
        rs_in.wait_recv()
        rs_send_buf[pl.ds(SQ, SQ), :] = (
            rs_send_buf[pl.ds(SQ, SQ), :] + rs_recv_buf[pl.ds(SQ, SQ), :])
        rs_out = pltpu.make_async_remote_copy(
            src_ref=rs_send_at(1), dst_ref=rs_recv_at(0),
            send_sem=rs_send_sems.at[1], recv_sem=rs_recv_sems.at[0],
            device_id=(right,), device_id_type=pl.DeviceIdType.MESH,
        )
        rs_out.start()
        rs_out.wait_recv()
        out_ref[:, :] = out_ref[:, :] + rs_recv_buf[pl.ds(0, SQ), :]
        rs_out.wait_send()

    out = pl.pallas_call(
        body,
        out_shape=jax.ShapeDtypeStruct((SQ, D), F32),
        in_specs=[
            pl.BlockSpec(memory_space=pltpu.MemorySpace.VMEM),
            pl.BlockSpec(memory_space=pltpu.MemorySpace.VMEM),
            pl.BlockSpec(memory_space=pltpu.MemorySpace.VMEM),
            pl.BlockSpec(memory_space=pl.ANY),
            pl.BlockSpec(memory_space=pl.ANY),
        ],
        out_specs=pl.BlockSpec(memory_space=pltpu.MemorySpace.VMEM),
        scratch_shapes=[
            pltpu.VMEM((HL, SKV, DH), BF16),
            pltpu.VMEM((HL, SKV, DH), BF16),
            pltpu.VMEM((HL, SKV, DH), F32),
            pltpu.VMEM((HL, SKV, DH), F32),
            pltpu.VMEM((2 * SQ, D), BF16),
            pltpu.VMEM((2 * SQ, D), BF16),
            pltpu.VMEM((2 * SQ, D), BF16),
            pltpu.VMEM((SQ, D), F32),
            pltpu.SemaphoreType.DMA((2 * HL,)),
            pltpu.SemaphoreType.DMA((2,)),
            pltpu.SemaphoreType.DMA((2,)),
            pltpu.SemaphoreType.DMA((2,)),
            pltpu.SemaphoreType.DMA((2,)),
            pltpu.SemaphoreType.REGULAR((2,)),
            pltpu.SemaphoreType.REGULAR((2,)),
        ],
        compiler_params=pltpu.CompilerParams(
            collective_id=0,
            vmem_limit_bytes=60 * 1024 * 1024,
        ),
    )(x2, wq3, wo3, K_ext, V_ext)
    return out.reshape(1, SQ, D)


# device time: 217497 ns/iter; 1.4001x vs baseline; 1.3040x over previous
import jax
import jax.numpy as jnp
from jax import lax
from jax.experimental import pallas as pl
from jax.experimental.pallas import tpu as pltpu

N_DEV = 8
SQ = 512
D = 1024
HL = 8
DH = 128
SKV = 2048
SCALE = 0.08838834764831843
F32 = jnp.float32
BF16 = jnp.bfloat16


def kernel(x, Wq, Wo, K_ext, V_ext):
    x2 = x.reshape(SQ, D).astype(BF16)
    wq3 = (Wq * SCALE).reshape(D, HL, DH).transpose(1, 0, 2).astype(BF16)
    wo3 = Wo.reshape(HL, DH, D).astype(BF16)

    def body(x_ref, wq_ref, wo_ref, k_hbm, v_hbm, out_ref,
             k_vmem, v_vmem, k_f32, v_f32, xslot, rs_send_buf, rs_recv_buf,
             pacc, kv_sems, x_send_sems, x_recv_sems, rs_send_sems,
             rs_recv_sems, x_credit, rs_credit):
        my = lax.axis_index("i")
        right = (my + 1) % N_DEV
        left = (my + N_DEV - 1) % N_DEV
        h0 = my * HL

        kv_copies = []
        for h in range(HL):
            kcp = pltpu.make_async_copy(
                k_hbm.at[0, :, h0 + h, :], k_f32.at[h], kv_sems.at[h])
            vcp = pltpu.make_async_copy(
                v_hbm.at[0, :, h0 + h, :], v_f32.at[h], kv_sems.at[HL + h])
            kcp.start()
            vcp.start()
            kv_copies.append((kcp, vcp))

        barrier = pltpu.get_barrier_semaphore()
        for nbr in (left, right):
            pl.semaphore_signal(barrier, inc=1, device_id=(nbr,),
                                device_id_type=pl.DeviceIdType.MESH)
        pl.semaphore_wait(barrier, 2)

        for kcp, vcp in kv_copies:
            kcp.wait()
            vcp.wait()
        k_vmem[:, :, :] = k_f32[:, :, :].astype(BF16)
        v_vmem[:, :, :] = v_f32[:, :, :].astype(BF16)

        def compute_chunk(src_ref, src_off, dst_ref, dst_off):
            pacc[:, :] = jnp.zeros((SQ, D), F32)

            def head_step(h, carry):
                xc = src_ref[pl.ds(src_off, SQ), :]
                qh = jnp.dot(xc, wq_ref[h], preferred_element_type=F32)
                s = lax.dot_general(
                    qh.astype(BF16), k_vmem[h], (((1,), (1,)), ((), ())),
                    preferred_element_type=F32)
                p = jnp.exp(s)
                l = jnp.sum(p, axis=1, keepdims=True)
                oh = jnp.dot(p.astype(BF16), v_vmem[h],
                             preferred_element_type=F32) / l
                pacc[:, :] = pacc[:, :] + jnp.dot(
                    oh.astype(BF16), wo_ref[h], preferred_element_type=F32)
                return carry

            lax.fori_loop(0, HL, head_step, None)
            dst_ref[pl.ds(dst_off, SQ), :] = pacc[:, :].astype(dst_ref.dtype)

        def xslot_at(p):
            return xslot.at[pl.ds(p * SQ, SQ), :]

        def rs_send_at(p):
            return rs_send_buf.at[pl.ds(p * SQ, SQ), :]

        def rs_recv_at(p):
            return rs_recv_buf.at[pl.ds(p * SQ, SQ), :]

        xslot[pl.ds(0, SQ), :] = x_ref[:, :]
        xr1 = pltpu.make_async_remote_copy(
            src_ref=xslot_at(0), dst_ref=xslot_at(1),
            send_sem=x_send_sems.at[0], recv_sem=x_recv_sems.at[1],
            device_id=(right,), device_id_type=pl.DeviceIdType.MESH,
        )
        xr1.start()
        compute_chunk(x_ref, 0, out_ref, 0)
        xr1.wait_send()
        pl.semaphore_signal(x_credit.at[0], inc=1, device_id=(left,),
                            device_id_type=pl.DeviceIdType.MESH)

        def step(t, _):
            sp = (t - 1) % 2
            dp = t % 2

            @pl.when(t >= 2)
            def _():
                @pl.when(t >= 4)
                def _():
                    pl.semaphore_wait(rs_credit.at[dp], 1)

                @pl.when(t >= 3)
                def _():
                    rs_in = pltpu.make_async_remote_copy(
                        src_ref=rs_send_at(sp),
                        dst_ref=rs_recv_at(sp),
                        send_sem=rs_send_sems.at[sp],
                        recv_sem=rs_recv_sems.at[sp],
                        device_id=(left,),
                        device_id_type=pl.DeviceIdType.MESH,
                    )
                    rs_in.wait_recv()
                    rs_send_buf[pl.ds(sp * SQ, SQ), :] = (
                        rs_send_buf[pl.ds(sp * SQ, SQ), :]
                        + rs_recv_buf[pl.ds(sp * SQ, SQ), :])
                    pl.semaphore_signal(rs_credit.at[sp], inc=1,
                                        device_id=(left,),
                                        device_id_type=pl.DeviceIdType.MESH)

                rs_out = pltpu.make_async_remote_copy(
                    src_ref=rs_send_at(sp),
                    dst_ref=rs_recv_at(dp),
                    send_sem=rs_send_sems.at[sp],
                    recv_sem=rs_recv_sems.at[dp],
                    device_id=(right,),
                    device_id_type=pl.DeviceIdType.MESH,
                )
                rs_out.start()

            x_in = pltpu.make_async_remote_copy(
                src_ref=xslot_at(sp),
                dst_ref=xslot_at(dp),
                send_sem=x_send_sems.at[dp],
                recv_sem=x_recv_sems.at[dp],
                device_id=(left,),
                device_id_type=pl.DeviceIdType.MESH,
            )
            x_in.wait_recv()

            @pl.when(t <= 6)
            def _():
                pl.semaphore_wait(x_credit.at[sp], 1)
                x_fwd = pltpu.make_async_remote_copy(
                    src_ref=xslot_at(dp),
                    dst_ref=xslot_at(sp),
                    send_sem=x_send_sems.at[dp],
                    recv_sem=x_recv_sems.at[sp],
                    device_id=(right,),
                    device_id_type=pl.DeviceIdType.MESH,
                )
                x_fwd.start()

            compute_chunk(xslot, dp * SQ, rs_send_buf, dp * SQ)

            @pl.when(t <= 6)
            def _():
                x_done = pltpu.make_async_remote_copy(
                    src_ref=xslot_at(dp),
                    dst_ref=xslot_at(sp),
                    send_sem=x_send_sems.at[dp],
                    recv_sem=x_recv_sems.at[sp],
                    device_id=(right,),
                    device_id_type=pl.DeviceIdType.MESH,
                )
                x_done.wait_send()

            @pl.when(t <= 5)
            def _():
                pl.semaphore_signal(x_credit.at[dp], inc=1,
                                    device_id=(left,),
                                    device_id_type=pl.DeviceIdType.MESH)

            @pl.when(t >= 2)
            def _():
                rs_done = pltpu.make_async_remote_copy(
                    src_ref=rs_send_at(sp),
                    dst_ref=rs_recv_at(dp),
                    send_sem=rs_send_sems.at[sp],
                    recv_sem=rs_recv_sems.at[dp],
                    device_id=(right,),
                    device_id_type=pl.DeviceIdType.MESH,
                )
                rs_done.wait_send()

            return _

        lax.fori_loop(1, N_DEV, step, None)

        pl.semaphore_wait(rs_credit.at[0], 1)
        rs_in = pltpu.make_async_remote_copy(
            src_ref=rs_send_at(1), dst_ref=rs_recv_at(1),
            send_sem=rs_send_sems.at[1], recv_sem=rs_recv_sems.at[1],
            device_id=(left,), device_id_type=pl.DeviceIdType.MESH,
        )
        rs_in.wait_recv()
        rs_send_buf[pl.ds(SQ, SQ), :] = (
            rs_send_buf[pl.ds(SQ, SQ), :] + rs_recv_buf[pl.ds(SQ, SQ), :])
        rs_out = pltpu.make_async_remote_copy(
            src_ref=rs_send_at(1), dst_ref=rs_recv_at(0),
            send_sem=rs_send_sems.at[1], recv_sem=rs_recv_sems.at[0],
            device_id=(right,), device_id_type=pl.DeviceIdType.MESH,
        )
        rs_out.start()
        rs_out.wait_recv()
        out_ref[:, :] = out_ref[:, :] + rs_recv_buf[pl.ds(0, SQ), :]
        rs_out.wait_send()

    out = pl.pallas_call(
        body,
        out_shape=jax.ShapeDtypeStruct((SQ, D), F32),
        in_specs=[
            pl.BlockSpec(memory_space=pltpu.MemorySpace.VMEM),
            pl.BlockSpec(memory_space=pltpu.MemorySpace.VMEM),
            pl.BlockSpec(memory_space=pltpu.MemorySpace.VMEM),
            pl.BlockSpec(memory_space=pl.ANY),
            pl.BlockSpec(memory_space=pl.ANY),
        ],
        out_specs=pl.BlockSpec(memory_space=pltpu.MemorySpace.VMEM),
        scratch_shapes=[
            pltpu.VMEM((HL, SKV, DH), BF16),
            pltpu.VMEM((HL, SKV, DH), BF16),
            pltpu.VMEM((HL, SKV, DH), F32),
            pltpu.VMEM((HL, SKV, DH), F32),
            pltpu.VMEM((2 * SQ, D), BF16),
            pltpu.VMEM((2 * SQ, D), BF16),
            pltpu.VMEM((2 * SQ, D), BF16),
            pltpu.VMEM((SQ, D), F32),
            pltpu.SemaphoreType.DMA((2 * HL,)),
            pltpu.SemaphoreType.DMA((2,)),
            pltpu.SemaphoreType.DMA((2,)),
            pltpu.SemaphoreType.DMA((2,)),
            pltpu.SemaphoreType.DMA((2,)),
            pltpu.SemaphoreType.REGULAR((2,)),
            pltpu.SemaphoreType.REGULAR((2,)),
        ],
        compiler_params=pltpu.CompilerParams(
            collective_id=0,
            vmem_limit_bytes=60 * 1024 * 1024,
        ),
    )(x2, wq3, wo3, K_ext, V_ext)
    return out.reshape(1, SQ, D)
